# baseline (speedup 1.0000x reference)
import jax
import jax.numpy as jnp
from jax.experimental import pallas as pl

_K = 1000
_BM = 2048


def _kmeans_block(x_ref, c_ref, dist_ref, label_ref):
    xb = x_ref[...]
    c = c_ref[...]
    b2 = jnp.sum(c * c, axis=1, keepdims=True)
    t = jax.lax.dot_general(
        c * -2.0, xb, (((1,), (1,)), ((), ())),
        preferred_element_type=jnp.float32) + b2
    ones8 = jnp.ones((8, xb.shape[1]), jnp.float32)
    a2 = jax.lax.dot_general(
        ones8, xb * xb, (((1,), (1,)), ((), ())),
        preferred_element_type=jnp.float32)[0]
    v = t[0:8, :]
    ri = jnp.zeros(v.shape, jnp.int32)
    for r in range(1, _K // 8):
        s = t[8 * r:8 * (r + 1), :]
        ri = jnp.where(s < v, r, ri)
        v = jnp.minimum(v, s)
    si = jax.lax.broadcasted_iota(jnp.int32, v.shape, 0)
    fullidx = ri * 8 + si
    m = jnp.min(v, axis=0)
    lbl = jnp.min(jnp.where(v == m[None, :], fullidx, 1 << 20), axis=0)
    label_ref[...] = lbl
    dist_ref[...] = jnp.sqrt(jnp.maximum(m + a2, 1e-12))


@jax.jit
def kernel(x, centers):
    n = x.shape[0]
    grid = (n // _BM,)
    dist, labels = pl.pallas_call(
        _kmeans_block,
        grid=grid,
        in_specs=[
            pl.BlockSpec((_BM, x.shape[1]), lambda i: (i, 0)),
            pl.BlockSpec((_K, centers.shape[1]), lambda i: (0, 0)),
        ],
        out_specs=[
            pl.BlockSpec((_BM,), lambda i: (i,)),
            pl.BlockSpec((_BM,), lambda i: (i,)),
        ],
        out_shape=[
            jax.ShapeDtypeStruct((n,), jnp.float32),
            jax.ShapeDtypeStruct((n,), jnp.int32),
        ],
    )(x, centers)
    return dist, labels


# speedup vs baseline: 1.0371x; 1.0371x over previous
import jax
import jax.numpy as jnp
from jax.experimental import pallas as pl
from jax.experimental.pallas import tpu as pltpu

_K = 1000
_BM = 4096


def _kmeans_block(x_ref, c_ref, dist_ref, label_ref):
    xb = x_ref[...]
    c = c_ref[...]
    b2 = jnp.sum(c * c, axis=1, keepdims=True)
    t = jax.lax.dot_general(
        c * -2.0, xb, (((1,), (1,)), ((), ())),
        preferred_element_type=jnp.float32) + b2
    ones8 = jnp.ones((8, xb.shape[1]), jnp.float32)
    a2 = jax.lax.dot_general(
        ones8, xb * xb, (((1,), (1,)), ((), ())),
        preferred_element_type=jnp.float32)[0]
    v = t[0:8, :]
    ri = jnp.zeros(v.shape, jnp.int32)
    for r in range(1, _K // 8):
        s = t[8 * r:8 * (r + 1), :]
        ri = jnp.where(s < v, r, ri)
        v = jnp.minimum(v, s)
    si = jax.lax.broadcasted_iota(jnp.int32, v.shape, 0)
    fullidx = ri * 8 + si
    m = jnp.min(v, axis=0)
    lbl = jnp.min(jnp.where(v == m[None, :], fullidx, 1 << 20), axis=0)
    label_ref[...] = lbl
    dist_ref[...] = jnp.sqrt(jnp.maximum(m + a2, 1e-12))


@jax.jit
def kernel(x, centers):
    n = x.shape[0]
    grid = (n // _BM,)
    dist, labels = pl.pallas_call(
        _kmeans_block,
        grid=grid,
        in_specs=[
            pl.BlockSpec((_BM, x.shape[1]), lambda i: (i, 0)),
            pl.BlockSpec((_K, centers.shape[1]), lambda i: (0, 0)),
        ],
        out_specs=[
            pl.BlockSpec((_BM,), lambda i: (i,)),
            pl.BlockSpec((_BM,), lambda i: (i,)),
        ],
        out_shape=[
            jax.ShapeDtypeStruct((n,), jnp.float32),
            jax.ShapeDtypeStruct((n,), jnp.int32),
        ],
        compiler_params=pltpu.CompilerParams(
            dimension_semantics=("parallel",)),
    )(x, centers)
    return dist, labels
